# baseline (speedup 1.0000x reference)
import jax
import jax.numpy as jnp
from jax.experimental import pallas as pl
from jax.experimental.pallas import tpu as pltpu


def _copy_body(x_ref, r_ref, d_ref, xo_ref, ro_ref, do_ref, sem_x, sem_r, sem_d):
    cx = pltpu.make_async_copy(x_ref, xo_ref, sem_x)
    cr = pltpu.make_async_copy(r_ref, ro_ref, sem_r)
    cd = pltpu.make_async_copy(d_ref, do_ref, sem_d)
    cx.start()
    cr.start()
    cd.start()
    cx.wait()
    cr.wait()
    cd.wait()


def kernel(sampled_point_xyz, sampled_point_ray_direction, sampled_point_distance):
    n = sampled_point_xyz.shape[0]
    xt = sampled_point_xyz.T
    rt = sampled_point_ray_direction.T
    pos_t, ray_t, dists = pl.pallas_call(
        _copy_body,
        in_specs=[pl.BlockSpec(memory_space=pl.ANY)] * 3,
        out_specs=[pl.BlockSpec(memory_space=pl.ANY)] * 3,
        out_shape=[
            jax.ShapeDtypeStruct((3, n), jnp.float32),
            jax.ShapeDtypeStruct((3, n), jnp.float32),
            jax.ShapeDtypeStruct((n,), jnp.float32),
        ],
        scratch_shapes=[pltpu.SemaphoreType.DMA] * 3,
    )(xt, rt, sampled_point_distance)
    return (pos_t.T, ray_t.T, dists)


# speedup vs baseline: 48.2139x; 48.2139x over previous
import jax
import jax.numpy as jnp
from jax.experimental import pallas as pl
from jax.experimental.pallas import tpu as pltpu

_G = 32


def _copy_body(x_ref, r_ref, d_ref, xo_ref, ro_ref, do_ref):
    xo_ref[...] = x_ref[...]
    ro_ref[...] = r_ref[...]
    do_ref[...] = d_ref[...]


def kernel(sampled_point_xyz, sampled_point_ray_direction, sampled_point_distance):
    n = sampled_point_xyz.shape[0]
    b = n // _G
    xt = sampled_point_xyz.T
    rt = sampled_point_ray_direction.T
    pos_t, ray_t, dists = pl.pallas_call(
        _copy_body,
        grid=(_G,),
        in_specs=[
            pl.BlockSpec((3, b), lambda i: (0, i)),
            pl.BlockSpec((3, b), lambda i: (0, i)),
            pl.BlockSpec((b,), lambda i: (i,)),
        ],
        out_specs=[
            pl.BlockSpec((3, b), lambda i: (0, i)),
            pl.BlockSpec((3, b), lambda i: (0, i)),
            pl.BlockSpec((b,), lambda i: (i,)),
        ],
        out_shape=[
            jax.ShapeDtypeStruct((3, n), jnp.float32),
            jax.ShapeDtypeStruct((3, n), jnp.float32),
            jax.ShapeDtypeStruct((n,), jnp.float32),
        ],
    )(xt, rt, sampled_point_distance)
    return (pos_t.T, ray_t.T, dists)


# speedup vs baseline: 48.8149x; 1.0125x over previous
import jax
import jax.numpy as jnp
from jax.experimental import pallas as pl
from jax.experimental.pallas import tpu as pltpu

_G = 16


def _copy_body(x_ref, r_ref, d_ref, xo_ref, ro_ref, do_ref):
    xo_ref[...] = x_ref[...]
    ro_ref[...] = r_ref[...]
    do_ref[...] = d_ref[...]


def kernel(sampled_point_xyz, sampled_point_ray_direction, sampled_point_distance):
    n = sampled_point_xyz.shape[0]
    b = n // _G
    xt = sampled_point_xyz.T
    rt = sampled_point_ray_direction.T
    pos_t, ray_t, dists = pl.pallas_call(
        _copy_body,
        grid=(_G,),
        in_specs=[
            pl.BlockSpec((3, b), lambda i: (0, i)),
            pl.BlockSpec((3, b), lambda i: (0, i)),
            pl.BlockSpec((b,), lambda i: (i,)),
        ],
        out_specs=[
            pl.BlockSpec((3, b), lambda i: (0, i)),
            pl.BlockSpec((3, b), lambda i: (0, i)),
            pl.BlockSpec((b,), lambda i: (i,)),
        ],
        out_shape=[
            jax.ShapeDtypeStruct((3, n), jnp.float32),
            jax.ShapeDtypeStruct((3, n), jnp.float32),
            jax.ShapeDtypeStruct((n,), jnp.float32),
        ],
    )(xt, rt, sampled_point_distance)
    return (pos_t.T, ray_t.T, dists)


# speedup vs baseline: 49.0162x; 1.0041x over previous
import jax
import jax.numpy as jnp
from jax.experimental import pallas as pl
from jax.experimental.pallas import tpu as pltpu

_G = 12


def _copy_body(x_ref, r_ref, d_ref, xo_ref, ro_ref, do_ref):
    xo_ref[...] = x_ref[...]
    ro_ref[...] = r_ref[...]
    do_ref[...] = d_ref[...]


def kernel(sampled_point_xyz, sampled_point_ray_direction, sampled_point_distance):
    n = sampled_point_xyz.shape[0]
    b = 349568
    bd = 350208
    xt = sampled_point_xyz.T
    rt = sampled_point_ray_direction.T
    pos_t, ray_t, dists = pl.pallas_call(
        _copy_body,
        grid=(_G,),
        in_specs=[
            pl.BlockSpec((3, b), lambda i: (0, i)),
            pl.BlockSpec((3, b), lambda i: (0, i)),
            pl.BlockSpec((bd,), lambda i: (i,)),
        ],
        out_specs=[
            pl.BlockSpec((3, b), lambda i: (0, i)),
            pl.BlockSpec((3, b), lambda i: (0, i)),
            pl.BlockSpec((bd,), lambda i: (i,)),
        ],
        out_shape=[
            jax.ShapeDtypeStruct((3, n), jnp.float32),
            jax.ShapeDtypeStruct((3, n), jnp.float32),
            jax.ShapeDtypeStruct((n,), jnp.float32),
        ],
    )(xt, rt, sampled_point_distance)
    return (pos_t.T, ray_t.T, dists)
